# Initial kernel scaffold; baseline (speedup 1.0000x reference)
import functools

import jax
import jax.numpy as jnp
import numpy as np
from jax import lax
from jax.experimental import pallas as pl
from jax.experimental.pallas import tpu as pltpu
from jax.experimental.pallas import tpu_sc as plsc

_B = 1024
_NBITS = 22
_NCODES = 23
_LTAB = 131072
_ROWW = 384
_NWORK = 32
_SPW = _B // _NWORK


def _jmap_word(wi):
    g, w = wi // 4, wi % 4
    return g if (g < 7 and w < 2) else g + 7


def _bit_weights():
    wmat = np.zeros((512, _NCODES), np.float32)
    for j in range(_NCODES):
        for i in range(_NBITS):
            wmat[6 + _NBITS * j + i, j] = float(1 << i)
    return jnp.asarray(wmat)


def _sign_expand():
    emat = np.zeros((_NCODES, 128), np.float32)
    for k in range(128):
        emat[_jmap_word(k // 2), k] = 1.0
    return jnp.asarray(emat)


def _decode_body(x_ref, w_ref, e_ref, codes_ref, sgn_ref):
    xf = x_ref[...].astype(jnp.float32)
    vals = jnp.dot(xf, w_ref[...], preferred_element_type=jnp.float32)
    codes_ref[...] = vals.astype(jnp.int32) & (_LTAB - 1)
    sgnv = jnp.where(vals > jnp.float32(_LTAB), -1.0, 1.0).astype(jnp.float32)
    sgn_ref[...] = jnp.dot(sgnv, e_ref[...], preferred_element_type=jnp.float32)


def _decode(x):
    return pl.pallas_call(
        _decode_body,
        out_shape=(
            jax.ShapeDtypeStruct((_B, _NCODES), jnp.int32),
            jax.ShapeDtypeStruct((_B, 128), jnp.float32),
        ),
    )(x, _bit_weights(), _sign_expand())


def _sc_gather_body(codes_hbm, dataw_hbm, out_hbm, idx_v, rows_v, asm_v, sem):
    cid = lax.axis_index("c")
    sid = lax.axis_index("s")
    wid = sid * 2 + cid
    base = wid * _SPW
    pltpu.sync_copy(codes_hbm.at[pl.ds(base, _SPW)], idx_v)

    lanes = lax.iota(jnp.int32, 16)
    jvecs, wvecs = [], []
    for q in range(4):
        wv = (q * 16 + lanes) % 4
        gv = (q * 16 + lanes) // 4
        jv = jnp.where((gv < 7) & (wv < 2), gv, gv + 7)
        jvecs.append(jv)
        wvecs.append(wv)

    def sample_body(i, carry):
        cp = pltpu.make_async_copy(dataw_hbm.at[idx_v.at[i]], rows_v, sem)
        cp.start()
        cp.wait()

        def t_body(tt, c2):
            t4 = tt * 32
            for k in range(8):
                for q in range(4):
                    vals = plsc.load_gather(
                        rows_v, [jvecs[q], wvecs[q] + (t4 + 4 * k)]
                    )
                    asm_v[tt * 8 + k, pl.ds(q * 16, 16)] = vals
            return c2

        lax.fori_loop(0, 12, t_body, 0, unroll=False)
        pltpu.sync_copy(asm_v, out_hbm.at[base + i])
        return carry

    lax.fori_loop(0, _SPW, sample_body, 0, unroll=False)


_sc_gather = functools.partial(
    pl.kernel,
    out_type=jax.ShapeDtypeStruct((_B, 96, 64), jnp.int32),
    mesh=plsc.VectorSubcoreMesh(core_axis_name="c", subcore_axis_name="s"),
    scratch_types=[
        pltpu.VMEM((_SPW, _NCODES), jnp.int32),
        pltpu.VMEM((_NCODES, _ROWW), jnp.int32),
        pltpu.VMEM((96, 64), jnp.int32),
        pltpu.SemaphoreType.DMA,
    ],
)(_sc_gather_body)


def _finish_body(g_ref, s_ref, o_ref):
    g = g_ref[...].astype(jnp.float32)
    s = s_ref[...][:, None, None, :]
    d = (g - 0.5).astype(jnp.float16).astype(jnp.float32)
    core = (0.5 + s * d).astype(jnp.float16).astype(jnp.float32)
    bs = g.shape[0]
    o_ref[:, :, 0:19, :] = jnp.full((bs, 2, 19, 128), 0.5, jnp.float32)
    o_ref[:, :, 19:67, :] = core
    o_ref[:, :, 67:126, :] = jnp.full((bs, 2, 59, 128), 0.5, jnp.float32)


def _finish(gathf, sgn):
    bs = 8
    return pl.pallas_call(
        _finish_body,
        grid=(_B // bs,),
        in_specs=[
            pl.BlockSpec((bs, 2, 48, 128), lambda i: (i, 0, 0, 0)),
            pl.BlockSpec((bs, 128), lambda i: (i, 0)),
        ],
        out_specs=pl.BlockSpec((bs, 2, 126, 128), lambda i: (i, 0, 0, 0)),
        out_shape=jax.ShapeDtypeStruct((_B, 2, 126, 128), jnp.float32),
    )(gathf, sgn)


def kernel(x, data):
    codes, sgn = _decode(x)
    dataw = lax.bitcast_convert_type(
        data.reshape(_LTAB, _ROWW, 2), jnp.int32
    )
    gathw = _sc_gather(codes, dataw)
    gathf = lax.bitcast_convert_type(gathw, jnp.float16).reshape(_B, 2, 48, 128)
    return _finish(gathf, sgn)


# speedup vs baseline: 1.5922x; 1.5922x over previous
import functools

import jax
import jax.numpy as jnp
import numpy as np
from jax import lax
from jax.experimental import pallas as pl
from jax.experimental.pallas import tpu as pltpu
from jax.experimental.pallas import tpu_sc as plsc

_B = 1024
_NBITS = 22
_NCODES = 23
_LTAB = 131072
_ROWW = 384
_NWORK = 32
_SPW = _B // _NWORK


def _jmap_word(wi):
    g, w = wi // 4, wi % 4
    return g if (g < 7 and w < 2) else g + 7


def _bit_weights():
    wmat = np.zeros((512, _NCODES), np.float32)
    for j in range(_NCODES):
        for i in range(_NBITS):
            wmat[6 + _NBITS * j + i, j] = float(1 << i)
    return jnp.asarray(wmat)


def _sign_expand():
    emat = np.zeros((_NCODES, 128), np.float32)
    for k in range(128):
        emat[_jmap_word(k // 2), k] = 1.0
    return jnp.asarray(emat)


def _decode_body(x_ref, w_ref, e_ref, codes_ref, sgn_ref):
    xf = x_ref[...].astype(jnp.float32)
    vals = jnp.dot(xf, w_ref[...], preferred_element_type=jnp.float32)
    codes_ref[...] = vals.astype(jnp.int32) & (_LTAB - 1)
    sgnv = jnp.where(vals > jnp.float32(_LTAB), -1.0, 1.0).astype(jnp.float32)
    sgn_ref[...] = jnp.dot(sgnv, e_ref[...], preferred_element_type=jnp.float32)


def _decode(x):
    return pl.pallas_call(
        _decode_body,
        out_shape=(
            jax.ShapeDtypeStruct((_B, _NCODES), jnp.int32),
            jax.ShapeDtypeStruct((_B, 128), jnp.float32),
        ),
    )(x, _bit_weights(), _sign_expand())


def _perm_table():
    tab = np.zeros((8, 16), np.int32)
    for q in range(4):
        ws = [q * 16 + ll for ll in range(16)]
        tab[q] = [_jmap_word(W) for W in ws]
        tab[4 + q] = [W % 4 for W in ws]
    return jnp.asarray(tab)


def _sc_gather_body(
    codes_hbm, dataw_hbm, perm_hbm, out_hbm, idx_v, perm_v, rows_v, asm_v, sem
):
    cid = lax.axis_index("c")
    sid = lax.axis_index("s")
    wid = sid * 2 + cid
    base = wid * _SPW
    pltpu.sync_copy(codes_hbm.at[pl.ds(base, _SPW)], idx_v)
    pltpu.sync_copy(perm_hbm, perm_v)

    def sample_body(i, carry):
        cp = pltpu.make_async_copy(dataw_hbm.at[idx_v.at[i]], rows_v, sem)
        cp.start()
        cp.wait()

        def t_body(tt, c2):
            t4 = tt * 32
            for k in range(8):
                for q in range(4):
                    vals = plsc.load_gather(
                        rows_v, [perm_v[q, :], perm_v[4 + q, :] + (t4 + 4 * k)]
                    )
                    asm_v[tt * 8 + k, pl.ds(q * 16, 16)] = vals
            return c2

        lax.fori_loop(0, 12, t_body, 0, unroll=False)
        pltpu.sync_copy(asm_v, out_hbm.at[base + i])
        return carry

    lax.fori_loop(0, _SPW, sample_body, 0, unroll=False)


@functools.cache
def _sc_gather():
    return pl.kernel(
        _sc_gather_body,
        out_type=jax.ShapeDtypeStruct((_B, 96, 64), jnp.int32),
        mesh=plsc.VectorSubcoreMesh(
            core_axis_name="c", subcore_axis_name="s", num_cores=2, num_subcores=16
        ),
        scratch_types=[
            pltpu.VMEM((_SPW, _NCODES), jnp.int32),
            pltpu.VMEM((8, 16), jnp.int32),
            pltpu.VMEM((_NCODES, _ROWW), jnp.int32),
            pltpu.VMEM((96, 64), jnp.int32),
            pltpu.SemaphoreType.DMA,
        ],
        compiler_params=pltpu.CompilerParams(
            needs_layout_passes=False, use_tc_tiling_on_sc=False
        ),
    )


def _finish_body(g_ref, s_ref, o_ref):
    g = g_ref[...]
    s = s_ref[...][:, None, :]
    core = 0.5 + s * (g - 0.5)
    bs = g.shape[0]
    half = jnp.full((bs, 1, 19, 128), 0.5, jnp.float32)
    o_ref[:, :, 0:19, :] = jnp.broadcast_to(half, (bs, 2, 19, 128))
    o_ref[:, 0, 19:67, :] = core[:, 0:48]
    o_ref[:, 1, 19:67, :] = core[:, 48:96]
    o_ref[:, :, 67:126, :] = jnp.full((bs, 2, 59, 128), 0.5, jnp.float32)


def _finish(gathf, sgn):
    bs = 8
    return pl.pallas_call(
        _finish_body,
        grid=(_B // bs,),
        in_specs=[
            pl.BlockSpec((bs, 96, 128), lambda i: (i, 0, 0)),
            pl.BlockSpec((bs, 128), lambda i: (i, 0)),
        ],
        out_specs=pl.BlockSpec((bs, 2, 126, 128), lambda i: (i, 0, 0, 0)),
        out_shape=jax.ShapeDtypeStruct((_B, 2, 126, 128), jnp.float32),
    )(gathf, sgn)


def kernel(x, data):
    codes, sgn = _decode(x)
    dataw = lax.bitcast_convert_type(
        data.reshape(_LTAB, _ROWW, 2), jnp.int32
    )
    gathw = _sc_gather()(codes, dataw, _perm_table())
    gathf = lax.bitcast_convert_type(gathw, jnp.float16).reshape(_B, 96, 128)
    return _finish(gathf.astype(jnp.float32), sgn)
